# Initial kernel scaffold; baseline (speedup 1.0000x reference)
import jax
import jax.numpy as jnp
from jax.experimental import pallas as pl


def kernel(actors, actor_idcs, lanes, lane_idcs, rpe_scene, rel_pose, W_rpe, Wq, Wk, Wv, Wo, ln1_g, ln1_b, W_ff1, b_ff1, W_ff2, b_ff2, ln2_g, ln2_b):
    raise NotImplementedError("write your pallas kernel here")


# speedup vs baseline: 2.0228x; 2.0228x over previous
import functools

import jax
import jax.numpy as jnp
import numpy as np
from jax.experimental import pallas as pl
from jax.experimental.pallas import tpu as pltpu

D = 256
H = 8
DH = D // H
N_AGENT = 256
N_MAP = 2048
KNN = 20
D_FF = 2048


def _fr_phase(theta):
    lane = jax.lax.iota(jnp.int32, 64).astype(jnp.float32)
    c = jnp.where(lane < 32, lane, lane - 32)
    fr = jnp.exp(c * (-2.0 / 64.0 * np.log(theta)))
    ph = jnp.where(lane < 32, np.float32(np.pi / 2), np.float32(0.0))
    return fr, ph


def _dense_body(actors_ref, x0_ref, x1_ref, th_ref, Wrpe_ref, Wq_ref,
                Wk_ref, Wv_ref, Wo_ref, ln1g_ref, ln1b_ref, Wf1_ref,
                bf1_ref, Wf2_ref, bf2_ref, ln2g_ref, ln2b_ref, out_ref):
    f32 = jnp.float32
    actors = actors_ref[...]
    x0 = x0_ref[...]
    x1 = x1_ref[...]
    th = th_ref[...]
    xc = jnp.cos(th)
    xs = jnp.sin(th)

    fr_pos, phase = _fr_phase(1000.0)
    fr_dir, _ = _fr_phase(10.0)

    def pe(x, fr):
        fr3 = fr[None, None, :].astype(f32)
        ph3 = phase[None, None, :].astype(f32)
        return jnp.sin(x[..., None] * fr3 + ph3)

    _rpe = jnp.concatenate(
        [pe(x0, fr_pos), pe(x1, fr_pos), pe(xc, fr_dir), pe(xs, fr_dir)],
        axis=-1)
    blk = x0.shape[0]
    rpe2 = _rpe.reshape(blk * KNN, D)

    Wrk = Wrpe_ref[...] @ Wk_ref[...]
    Wrv = Wrpe_ref[...] @ Wv_ref[...]
    Rk = (rpe2 @ Wrk).reshape(blk, KNN, H, DH)
    q4 = (actors @ Wq_ref[...]).reshape(blk, 1, H, DH)
    logits = (q4 * Rk).sum(axis=-1) * (1.0 / np.sqrt(DH))
    m = logits.max(axis=1, keepdims=True)
    p = jnp.exp(logits - m)
    attn = p / p.sum(axis=1, keepdims=True)

    Rv = (rpe2 @ Wrv).reshape(blk, KNN, H, DH)
    ctx = (attn[..., None] * Rv).sum(axis=1).reshape(blk, D)
    ctx = ctx + actors @ Wv_ref[...]

    def ln(x, g, b):
        mu = jnp.mean(x, axis=-1, keepdims=True)
        var = jnp.mean((x - mu) ** 2, axis=-1, keepdims=True)
        return (x - mu) / jnp.sqrt(var + 1e-5) * g + b

    x = ln(actors + ctx @ Wo_ref[...], ln1g_ref[...], ln1b_ref[...])
    ff = jnp.maximum(x @ Wf1_ref[...] + bf1_ref[...], 0.0) @ Wf2_ref[...]
    ff = ff + bf2_ref[...]
    out_ref[...] = ln(x + ff, ln2g_ref[...], ln2b_ref[...])


_BLK = 64


def _fixed(shape):
    return pl.BlockSpec(shape, lambda i: tuple(0 for _ in shape))


@jax.jit
def _dense_block(actors, x0, x1, th, W_rpe, Wq, Wk, Wv, Wo, ln1_g, ln1_b,
                 W_ff1, b_ff1, W_ff2, b_ff2, ln2_g, ln2_b):
    nblk = N_AGENT // _BLK
    row_spec = pl.BlockSpec((_BLK, D), lambda i: (i, 0))
    knn_spec = pl.BlockSpec((_BLK, KNN), lambda i: (i, 0))
    return pl.pallas_call(
        _dense_body,
        grid=(nblk,),
        in_specs=[row_spec, knn_spec, knn_spec, knn_spec,
                  _fixed((D, D)), _fixed((D, D)), _fixed((D, D)),
                  _fixed((D, D)), _fixed((D, D)),
                  _fixed((1, D)), _fixed((1, D)),
                  _fixed((D, D_FF)), _fixed((1, D_FF)),
                  _fixed((D_FF, D)), _fixed((1, D)),
                  _fixed((1, D)), _fixed((1, D))],
        out_specs=row_spec,
        out_shape=jax.ShapeDtypeStruct((N_AGENT, D), jnp.float32),
    )(actors, x0, x1, th, W_rpe, Wq, Wk, Wv, Wo,
      ln1_g.reshape(1, D), ln1_b.reshape(1, D),
      W_ff1, b_ff1.reshape(1, D_FF), W_ff2, b_ff2.reshape(1, D),
      ln2_g.reshape(1, D), ln2_b.reshape(1, D))


def kernel(actors, actor_idcs, lanes, lane_idcs, rpe_scene, rel_pose,
           W_rpe, Wq, Wk, Wv, Wo, ln1_g, ln1_b, W_ff1, b_ff1, W_ff2,
           b_ff2, ln2_g, ln2_b):
    rd = rpe_scene[2, :N_AGENT, N_AGENT:]
    _, idx = jax.lax.top_k(-rd, KNN)
    rp = rel_pose[:N_AGENT, N_AGENT:, :]
    rpe = jnp.take_along_axis(rp, idx[..., None], axis=1)
    x = _dense_block(actors, rpe[..., 0], rpe[..., 1], rpe[..., 2],
                     W_rpe, Wq, Wk, Wv, Wo, ln1_g, ln1_b,
                     W_ff1, b_ff1, W_ff2, b_ff2, ln2_g, ln2_b)
    return (x, lanes)
